# baseline (speedup 1.0000x reference)
import numpy as np
import jax
import jax.numpy as jnp
from jax.experimental import pallas as pl
from jax.experimental.pallas import tpu as pltpu

_N = 262144
_CHUNKS = (65536, 65536, 65536, 32768, 24576, 8192)
_STARTS = tuple(int(s) for s in np.cumsum((0,) + _CHUNKS[:-1]))
_NCH = len(_CHUNKS)
_CMAX = max(_CHUNKS)
_OSLOTS = 3


def _offcol():
    corners = [(i, j, k) for i in (0, 1) for j in (0, 1) for k in (0, 1)]
    off = np.zeros((24, 1), np.int32)
    for d in range(3):
        for c in range(8):
            off[d * 8 + c, 0] = corners[c][d]
    return off


_OFF_NP = _offcol()


def _body(xt_hbm, minb_ref, rng_ref, nl_ref, off_ref, xnt_hbm, box_hbm,
          xin, xnb, boxb, in_sem, xn_sem, box_sem):
    def in_cp(c, slot):
        sz = _CHUNKS[c]
        return pltpu.make_async_copy(
            xt_hbm.at[:, pl.ds(_STARTS[c], sz)],
            xin.at[slot, :, pl.ds(0, sz)], in_sem.at[slot])

    def xn_cp(c, slot):
        sz = _CHUNKS[c]
        return pltpu.make_async_copy(
            xnb.at[slot, :, pl.ds(0, sz)],
            xnt_hbm.at[:, pl.ds(_STARTS[c], sz)], xn_sem.at[slot])

    def box_cp(c, slot):
        sz = _CHUNKS[c]
        return pltpu.make_async_copy(
            boxb.at[slot, :, pl.ds(0, sz)],
            box_hbm.at[:, pl.ds(_STARTS[c], sz)], box_sem.at[slot])

    in_cp(0, 0).start()
    for c in range(_NCH):
        islot = c & 1
        oslot = c % _OSLOTS
        if c + 1 < _NCH:
            in_cp(c + 1, 1 - islot).start()
        in_cp(c, islot).wait()
        if c >= _OSLOTS:
            xn_cp(c - _OSLOTS, oslot).wait()
            box_cp(c - _OSLOTS, oslot).wait()
        sz = _CHUNKS[c]
        x = xin[islot, :, :sz]
        xn = (x - minb_ref[...]) / rng_ref[...]
        xnb[oslot, :, :sz] = xn
        mbi = jnp.floor(xn * nl_ref[0, 0]).astype(jnp.int32)
        for d in range(3):
            boxb[oslot, d * 8:(d + 1) * 8, :sz] = (
                jnp.broadcast_to(mbi[d:d + 1, :], (8, sz))
                + off_ref[d * 8:(d + 1) * 8, :])
        xn_cp(c, oslot).start()
        box_cp(c, oslot).start()
    for c in range(max(0, _NCH - _OSLOTS), _NCH):
        oslot = c % _OSLOTS
        xn_cp(c, oslot).wait()
        box_cp(c, oslot).wait()


@jax.jit
def kernel(x, bounding_box, tables):
    del tables
    min_res = jnp.array([16.0], dtype=jnp.float32)
    max_res = jnp.array([512.0], dtype=jnp.float32)
    b = jnp.exp((jnp.log(max_res) - jnp.log(min_res)) / 15)
    nl = jnp.floor(min_res * b ** 15).reshape(1, 1)

    minb = bounding_box[0].reshape(3, 1)
    rng = (bounding_box[1] - bounding_box[0]).reshape(3, 1)
    off = jnp.asarray(_OFF_NP)

    xt = x.T
    hbm = pl.BlockSpec(memory_space=pltpu.MemorySpace.HBM)
    vmem = pl.BlockSpec(memory_space=pltpu.MemorySpace.VMEM)
    xnt, box24 = pl.pallas_call(
        _body,
        in_specs=[hbm, vmem, vmem, vmem, vmem],
        out_specs=[hbm, hbm],
        out_shape=[
            jax.ShapeDtypeStruct((3, _N), jnp.float32),
            jax.ShapeDtypeStruct((24, _N), jnp.int32),
        ],
        scratch_shapes=[
            pltpu.VMEM((2, 3, _CMAX), jnp.float32),
            pltpu.VMEM((_OSLOTS, 3, _CMAX), jnp.float32),
            pltpu.VMEM((_OSLOTS, 24, _CMAX), jnp.int32),
            pltpu.SemaphoreType.DMA((2,)),
            pltpu.SemaphoreType.DMA((_OSLOTS,)),
            pltpu.SemaphoreType.DMA((_OSLOTS,)),
        ],
    )(xt, minb, rng, nl, off)
    box = box24.reshape(3, 8, _N).transpose(2, 1, 0)
    return xnt.T, box


# speedup vs baseline: 1.0529x; 1.0529x over previous
import numpy as np
import jax
import jax.numpy as jnp
from jax.experimental import pallas as pl
from jax.experimental.pallas import tpu as pltpu

_N = 262144
_CHUNKS = (32768,) * 8
_STARTS = tuple(int(s) for s in np.cumsum((0,) + _CHUNKS[:-1]))
_NCH = len(_CHUNKS)
_CMAX = max(_CHUNKS)
_OSLOTS = 4


def _offcol():
    corners = [(i, j, k) for i in (0, 1) for j in (0, 1) for k in (0, 1)]
    off = np.zeros((24, 1), np.int32)
    for d in range(3):
        for c in range(8):
            off[d * 8 + c, 0] = corners[c][d]
    return off


_OFF_NP = _offcol()


def _body(xt_hbm, minb_ref, rng_ref, nl_ref, off_ref, xnt_hbm, box_hbm,
          xin, xnb, boxb, in_sem, xn_sem, box_sem):
    def in_cp(c, slot):
        sz = _CHUNKS[c]
        return pltpu.make_async_copy(
            xt_hbm.at[:, pl.ds(_STARTS[c], sz)],
            xin.at[slot, :, pl.ds(0, sz)], in_sem.at[slot])

    def xn_cp(c, slot):
        sz = _CHUNKS[c]
        return pltpu.make_async_copy(
            xnb.at[slot, :, pl.ds(0, sz)],
            xnt_hbm.at[:, pl.ds(_STARTS[c], sz)], xn_sem.at[slot])

    def box_cp(c, slot):
        sz = _CHUNKS[c]
        return pltpu.make_async_copy(
            boxb.at[slot, :, pl.ds(0, sz)],
            box_hbm.at[:, pl.ds(_STARTS[c], sz)], box_sem.at[slot])

    in_cp(0, 0).start()
    for c in range(_NCH):
        islot = c & 1
        oslot = c % _OSLOTS
        if c + 1 < _NCH:
            in_cp(c + 1, 1 - islot).start()
        in_cp(c, islot).wait()
        if c >= _OSLOTS:
            xn_cp(c - _OSLOTS, oslot).wait()
            box_cp(c - _OSLOTS, oslot).wait()
        sz = _CHUNKS[c]
        x = xin[islot, :, :sz]
        xn = (x - minb_ref[...]) / rng_ref[...]
        xnb[oslot, :, :sz] = xn
        mbi = jnp.floor(xn * nl_ref[0, 0]).astype(jnp.int32)
        for d in range(3):
            boxb[oslot, d * 8:(d + 1) * 8, :sz] = (
                jnp.broadcast_to(mbi[d:d + 1, :], (8, sz))
                + off_ref[d * 8:(d + 1) * 8, :])
        xn_cp(c, oslot).start()
        box_cp(c, oslot).start()
    for c in range(max(0, _NCH - _OSLOTS), _NCH):
        oslot = c % _OSLOTS
        xn_cp(c, oslot).wait()
        box_cp(c, oslot).wait()


@jax.jit
def kernel(x, bounding_box, tables):
    del tables
    min_res = jnp.array([16.0], dtype=jnp.float32)
    max_res = jnp.array([512.0], dtype=jnp.float32)
    b = jnp.exp((jnp.log(max_res) - jnp.log(min_res)) / 15)
    nl = jnp.floor(min_res * b ** 15).reshape(1, 1)

    minb = bounding_box[0].reshape(3, 1)
    rng = (bounding_box[1] - bounding_box[0]).reshape(3, 1)
    off = jnp.asarray(_OFF_NP)

    xt = x.T
    hbm = pl.BlockSpec(memory_space=pltpu.MemorySpace.HBM)
    vmem = pl.BlockSpec(memory_space=pltpu.MemorySpace.VMEM)
    xnt, box24 = pl.pallas_call(
        _body,
        in_specs=[hbm, vmem, vmem, vmem, vmem],
        out_specs=[hbm, hbm],
        out_shape=[
            jax.ShapeDtypeStruct((3, _N), jnp.float32),
            jax.ShapeDtypeStruct((24, _N), jnp.int32),
        ],
        scratch_shapes=[
            pltpu.VMEM((2, 3, _CMAX), jnp.float32),
            pltpu.VMEM((_OSLOTS, 3, _CMAX), jnp.float32),
            pltpu.VMEM((_OSLOTS, 24, _CMAX), jnp.int32),
            pltpu.SemaphoreType.DMA((2,)),
            pltpu.SemaphoreType.DMA((_OSLOTS,)),
            pltpu.SemaphoreType.DMA((_OSLOTS,)),
        ],
    )(xt, minb, rng, nl, off)
    box = box24.reshape(3, 8, _N).transpose(2, 1, 0)
    return xnt.T, box


# speedup vs baseline: 1.0582x; 1.0050x over previous
import numpy as np
import jax
import jax.numpy as jnp
from jax.experimental import pallas as pl
from jax.experimental.pallas import tpu as pltpu

_N = 262144
_BLK = 65536


def _offcol():
    corners = [(i, j, k) for i in (0, 1) for j in (0, 1) for k in (0, 1)]
    off = np.zeros((24, 1), np.int32)
    for d in range(3):
        for c in range(8):
            off[d * 8 + c, 0] = corners[c][d]
    return off


_OFF_NP = _offcol()


def _body(xt_ref, minb_ref, rng_ref, nl_ref, off_ref, xnt_ref, box_ref):
    xt = xt_ref[...]
    xn = (xt - minb_ref[...]) / rng_ref[...]
    xnt_ref[...] = xn
    mbi = jnp.floor(xn * nl_ref[0, 0]).astype(jnp.int32)
    for d in range(3):
        box_ref[d * 8:(d + 1) * 8, :] = (
            jnp.broadcast_to(mbi[d:d + 1, :], (8, _BLK))
            + off_ref[d * 8:(d + 1) * 8, :])


@jax.jit
def kernel(x, bounding_box, tables):
    del tables
    min_res = jnp.array([16.0], dtype=jnp.float32)
    max_res = jnp.array([512.0], dtype=jnp.float32)
    b = jnp.exp((jnp.log(max_res) - jnp.log(min_res)) / 15)
    nl = jnp.floor(min_res * b ** 15).reshape(1, 1)

    minb = bounding_box[0].reshape(3, 1)
    rng = (bounding_box[1] - bounding_box[0]).reshape(3, 1)
    off = jnp.asarray(_OFF_NP)

    xt = x.T
    grid = (_N // _BLK,)
    xnt, box24 = pl.pallas_call(
        _body,
        grid=grid,
        in_specs=[
            pl.BlockSpec((3, _BLK), lambda i: (0, i)),
            pl.BlockSpec((3, 1), lambda i: (0, 0)),
            pl.BlockSpec((3, 1), lambda i: (0, 0)),
            pl.BlockSpec((1, 1), lambda i: (0, 0)),
            pl.BlockSpec((24, 1), lambda i: (0, 0)),
        ],
        out_specs=[
            pl.BlockSpec((3, _BLK), lambda i: (0, i)),
            pl.BlockSpec((24, _BLK), lambda i: (0, i)),
        ],
        out_shape=[
            jax.ShapeDtypeStruct((3, _N), jnp.float32),
            jax.ShapeDtypeStruct((24, _N), jnp.int32),
        ],
    )(xt, minb, rng, nl, off)
    box = box24.reshape(3, 8, _N).transpose(2, 1, 0)
    return xnt.T, box
